# baseline (speedup 1.0000x reference)
import functools

import jax
import jax.numpy as jnp
from jax import lax
from jax.experimental import pallas as pl
from jax.experimental.pallas import tpu as pltpu
from jax.experimental.pallas import tpu_sc as plsc

BATCH = 16384
DIM = 64
NUM_CORES = 2
NUM_SUBCORES = 16
NUM_WORKERS = NUM_CORES * NUM_SUBCORES
ROWS_PER_WORKER = BATCH // NUM_WORKERS
CHUNK = 128
NCHUNK = ROWS_PER_WORKER // CHUNK

COPY_GRID = 8
COPY_BLOCK = BATCH // COPY_GRID


def _gather_body(table_hbm, idx_hbm, out_hbm, idx_v, rows_v, sem):
    wid = lax.axis_index("s") * NUM_CORES + lax.axis_index("c")
    base = wid * ROWS_PER_WORKER
    pltpu.sync_copy(idx_hbm.at[pl.ds(base, ROWS_PER_WORKER)], idx_v)
    gathers = [
        pltpu.async_copy(
            table_hbm.at[idx_v.at[pl.ds(j * CHUNK, CHUNK)]],
            rows_v.at[pl.ds(j * CHUNK, CHUNK)],
            sem,
        )
        for j in range(NCHUNK)
    ]
    for g in gathers:
        g.wait()
    pltpu.sync_copy(rows_v, out_hbm.at[pl.ds(base, ROWS_PER_WORKER)])


_gather = functools.partial(
    pl.kernel,
    out_type=jax.ShapeDtypeStruct((BATCH, DIM), jnp.float32),
    mesh=plsc.VectorSubcoreMesh(core_axis_name="c", subcore_axis_name="s"),
    scratch_types=[
        pltpu.VMEM((ROWS_PER_WORKER,), jnp.int32),
        pltpu.VMEM((ROWS_PER_WORKER, DIM), jnp.float32),
        pltpu.SemaphoreType.DMA,
    ],
    compiler_params=pltpu.CompilerParams(use_tc_tiling_on_sc=False),
)(_gather_body)


def _copy_body(sub_ref, obj_ref, sub_out_ref, obj_out_ref):
    sub_out_ref[...] = sub_ref[...]
    obj_out_ref[...] = obj_ref[...]


_passthrough = pl.pallas_call(
    _copy_body,
    grid=(COPY_GRID,),
    in_specs=[
        pl.BlockSpec((COPY_BLOCK, DIM), lambda i: (i, 0)),
        pl.BlockSpec((COPY_BLOCK, DIM), lambda i: (i, 0)),
    ],
    out_specs=[
        pl.BlockSpec((COPY_BLOCK, DIM), lambda i: (i, 0)),
        pl.BlockSpec((COPY_BLOCK, DIM), lambda i: (i, 0)),
    ],
    out_shape=(
        jax.ShapeDtypeStruct((BATCH, DIM), jnp.float32),
        jax.ShapeDtypeStruct((BATCH, DIM), jnp.float32),
    ),
)


def kernel(subject_embeddings, relation_ids, object_embeddings, relation_table):
    relation_embeddings = _gather(relation_table, relation_ids.astype(jnp.int32))
    sub_out, obj_out = _passthrough(subject_embeddings, object_embeddings)
    return (sub_out, relation_embeddings, obj_out)


# speedup vs baseline: 1.7079x; 1.7079x over previous
import functools

import jax
import jax.numpy as jnp
from jax import lax
from jax.experimental import pallas as pl
from jax.experimental.pallas import tpu as pltpu
from jax.experimental.pallas import tpu_sc as plsc

BATCH = 16384
DIM = 64
NUM_CORES = 2
NUM_SUBCORES = 16
NUM_WORKERS = NUM_CORES * NUM_SUBCORES
ROWS_PER_WORKER = BATCH // NUM_WORKERS
CHUNK = 128
NCHUNK = ROWS_PER_WORKER // CHUNK

COPY_GRID = 8
COPY_BLOCK = BATCH // COPY_GRID


def _gather_body(table_hbm, idx_hbm, out_hbm, idx_v, rows_v, sem):
    wid = lax.axis_index("s") * NUM_CORES + lax.axis_index("c")
    base = wid * ROWS_PER_WORKER
    pltpu.sync_copy(idx_hbm.at[pl.ds(wid * NCHUNK, NCHUNK)], idx_v)
    gathers = [
        pltpu.async_copy(
            table_hbm.at[idx_v.at[j]],
            rows_v.at[pl.ds(j * CHUNK, CHUNK)],
            sem,
        )
        for j in range(NCHUNK)
    ]
    for g in gathers:
        g.wait()
    pltpu.sync_copy(rows_v, out_hbm.at[pl.ds(base, ROWS_PER_WORKER)])


_gather = functools.partial(
    pl.kernel,
    out_type=jax.ShapeDtypeStruct((BATCH, DIM), jnp.float32),
    mesh=plsc.VectorSubcoreMesh(core_axis_name="c", subcore_axis_name="s"),
    scratch_types=[
        pltpu.VMEM((NCHUNK, CHUNK), jnp.int32),
        pltpu.VMEM((ROWS_PER_WORKER, DIM), jnp.float32),
        pltpu.SemaphoreType.DMA,
    ],
    compiler_params=pltpu.CompilerParams(use_tc_tiling_on_sc=False),
)(_gather_body)


def _copy_body(sub_ref, obj_ref, sub_out_ref, obj_out_ref):
    sub_out_ref[...] = sub_ref[...]
    obj_out_ref[...] = obj_ref[...]


_passthrough_t = pl.pallas_call(
    _copy_body,
    grid=(COPY_GRID,),
    in_specs=[
        pl.BlockSpec((DIM, COPY_BLOCK), lambda i: (0, i)),
        pl.BlockSpec((DIM, COPY_BLOCK), lambda i: (0, i)),
    ],
    out_specs=[
        pl.BlockSpec((DIM, COPY_BLOCK), lambda i: (0, i)),
        pl.BlockSpec((DIM, COPY_BLOCK), lambda i: (0, i)),
    ],
    out_shape=(
        jax.ShapeDtypeStruct((DIM, BATCH), jnp.float32),
        jax.ShapeDtypeStruct((DIM, BATCH), jnp.float32),
    ),
)


def kernel(subject_embeddings, relation_ids, object_embeddings, relation_table):
    idx2d = relation_ids.astype(jnp.int32).reshape(NUM_WORKERS * NCHUNK, CHUNK)
    relation_embeddings = _gather(relation_table, idx2d)
    sub_t, obj_t = _passthrough_t(subject_embeddings.T, object_embeddings.T)
    return (sub_t.T, relation_embeddings, obj_t.T)
